# baseline (speedup 1.0000x reference)
import functools

import jax
import jax.numpy as jnp
from jax import lax
from jax.experimental import pallas as pl
from jax.experimental.pallas import tpu as pltpu
from jax.experimental.pallas import tpu_sc as plsc

_VOCAB = 100000
_EMBED_DIM = 64
_BATCH = 4096
_SEQ_LEN = 200

_NUM_WORKERS = 32
_BBLK = _BATCH // _NUM_WORKERS

_mesh = plsc.VectorSubcoreMesh(core_axis_name="c", subcore_axis_name="s")


@functools.partial(
    pl.kernel,
    mesh=_mesh,
    out_type=jax.ShapeDtypeStruct((_SEQ_LEN, 8, _NUM_WORKERS, 8, 128), jnp.float32),
    scratch_types=[
        pltpu.VMEM((_SEQ_LEN, _BBLK), jnp.int32),
        pltpu.VMEM((_BBLK, _EMBED_DIM), jnp.float32),
        pltpu.VMEM((_BBLK, _EMBED_DIM), jnp.float32),
        pltpu.VMEM((8, 8, 128), jnp.float32),
        pltpu.VMEM((8, 8, 128), jnp.float32),
        pltpu.SemaphoreType.DMA,
        pltpu.SemaphoreType.DMA,
        pltpu.SemaphoreType.DMA,
        pltpu.SemaphoreType.DMA,
    ],
    compiler_params=pltpu.CompilerParams(
        use_tc_tiling_on_sc=False, needs_layout_passes=False
    ),
)
def _embedding_gather(idx_hbm, table_hbm, out_hbm,
                      idx_v, rows0, rows1, tbuf0, tbuf1,
                      gsem0, gsem1, osem0, osem1):
    wid = lax.axis_index("s") * 2 + lax.axis_index("c")
    pltpu.sync_copy(idx_hbm.at[wid], idx_v)

    lane = jax.lax.iota(jnp.int32, 16)
    lanes = [lane + (16 * bb) for bb in range(8)]

    def transpose_tile(rows, tbuf):
        for d in range(_EMBED_DIM):
            col = jnp.full((16,), d, jnp.int32)
            for bb in range(8):
                vals = plsc.load_gather(rows, [lanes[bb], col])
                tbuf[d // 8, d % 8, pl.ds(16 * bb, 16)] = vals

    def gather(l, rows, sem):
        pltpu.async_copy(table_hbm.at[idx_v.at[l]], rows, sem)

    def gather_wait(l, rows, sem):
        pltpu.make_async_copy(table_hbm.at[idx_v.at[l]], rows, sem).wait()

    def store(l, tbuf, sem):
        pltpu.async_copy(tbuf, out_hbm.at[l, :, wid], sem)

    def store_wait(l, tbuf, sem):
        pltpu.make_async_copy(tbuf, out_hbm.at[l, :, wid], sem).wait()

    gather(0, rows0, gsem0)
    gather(1, rows1, gsem1)

    def body(i, carry):
        l = 2 * i
        gather_wait(l, rows0, gsem0)

        @pl.when(i > 0)
        def _():
            store_wait(l - 2, tbuf0, osem0)
        transpose_tile(rows0, tbuf0)
        store(l, tbuf0, osem0)

        @pl.when(l + 2 < _SEQ_LEN)
        def _():
            gather(l + 2, rows0, gsem0)

        gather_wait(l + 1, rows1, gsem1)

        @pl.when(i > 0)
        def _():
            store_wait(l - 1, tbuf1, osem1)
        transpose_tile(rows1, tbuf1)
        store(l + 1, tbuf1, osem1)

        @pl.when(l + 3 < _SEQ_LEN)
        def _():
            gather(l + 3, rows1, gsem1)

        return carry

    lax.fori_loop(0, _SEQ_LEN // 2, body, 0)
    store_wait(_SEQ_LEN - 2, tbuf0, osem0)
    store_wait(_SEQ_LEN - 1, tbuf1, osem1)


def kernel(indices, embedding_matrix):
    idx = indices.astype(jnp.int32).reshape(_NUM_WORKERS, _BBLK, _SEQ_LEN)
    idx = idx.transpose(0, 2, 1)
    out = _embedding_gather(idx, embedding_matrix)
    return out.transpose(2, 4, 0, 1, 3).reshape(_BATCH, _SEQ_LEN, _EMBED_DIM)


# speedup vs baseline: 1.9867x; 1.9867x over previous
import functools

import jax
import jax.numpy as jnp
from jax import lax
from jax.experimental import pallas as pl
from jax.experimental.pallas import tpu as pltpu
from jax.experimental.pallas import tpu_sc as plsc

_VOCAB = 100000
_EMBED_DIM = 64
_BATCH = 4096
_SEQ_LEN = 200

_NUM_WORKERS = 32
_BBLK = _BATCH // _NUM_WORKERS

_mesh = plsc.VectorSubcoreMesh(core_axis_name="c", subcore_axis_name="s")


@functools.partial(
    pl.kernel,
    mesh=_mesh,
    out_type=jax.ShapeDtypeStruct((_SEQ_LEN, 8, _NUM_WORKERS, 8, 128), jnp.float32),
    scratch_types=[
        pltpu.VMEM((_SEQ_LEN, _BBLK), jnp.int32),
        pltpu.VMEM((_BBLK, _EMBED_DIM), jnp.float32),
        pltpu.VMEM((_BBLK, _EMBED_DIM), jnp.float32),
        pltpu.VMEM((8, 8, 128), jnp.float32),
        pltpu.VMEM((8, 8, 128), jnp.float32),
        pltpu.SemaphoreType.DMA,
        pltpu.SemaphoreType.DMA,
        pltpu.SemaphoreType.DMA,
        pltpu.SemaphoreType.DMA,
    ],
    compiler_params=pltpu.CompilerParams(
        use_tc_tiling_on_sc=False, needs_layout_passes=False
    ),
)
def _embedding_gather(idx_hbm, table_hbm, out_hbm,
                      idx_v, rows0, rows1, tbuf0, tbuf1,
                      gsem0, gsem1, osem0, osem1):
    wid = lax.axis_index("s") * 2 + lax.axis_index("c")
    pltpu.sync_copy(idx_hbm.at[wid], idx_v)

    lane = jax.lax.iota(jnp.int32, 16)
    lanes = [lane + (16 * bb) for bb in range(8)]

    def transpose_tile(rows, tbuf):
        @plsc.parallel_loop(0, _EMBED_DIM, 1, unroll=8)
        def dbody(d):
            col = jnp.full((16,), d, jnp.int32)
            for bb in range(8):
                vals = plsc.load_gather(rows, [lanes[bb], col])
                tbuf[d // 8, d % 8, pl.ds(16 * bb, 16)] = vals

    def gather(l, rows, sem):
        pltpu.async_copy(table_hbm.at[idx_v.at[l]], rows, sem)

    def gather_wait(l, rows, sem):
        pltpu.make_async_copy(table_hbm.at[idx_v.at[l]], rows, sem).wait()

    def store(l, tbuf, sem):
        pltpu.async_copy(tbuf, out_hbm.at[l, :, wid], sem)

    def store_wait(l, tbuf, sem):
        pltpu.make_async_copy(tbuf, out_hbm.at[l, :, wid], sem).wait()

    gather(0, rows0, gsem0)
    gather(1, rows1, gsem1)

    def body(i, carry):
        l = 2 * i
        gather_wait(l, rows0, gsem0)

        @pl.when(i > 0)
        def _():
            store_wait(l - 2, tbuf0, osem0)
        transpose_tile(rows0, tbuf0)
        store(l, tbuf0, osem0)

        @pl.when(l + 2 < _SEQ_LEN)
        def _():
            gather(l + 2, rows0, gsem0)

        gather_wait(l + 1, rows1, gsem1)

        @pl.when(i > 0)
        def _():
            store_wait(l - 1, tbuf1, osem1)
        transpose_tile(rows1, tbuf1)
        store(l + 1, tbuf1, osem1)

        @pl.when(l + 3 < _SEQ_LEN)
        def _():
            gather(l + 3, rows1, gsem1)

        return carry

    lax.fori_loop(0, _SEQ_LEN // 2, body, 0)
    store_wait(_SEQ_LEN - 2, tbuf0, osem0)
    store_wait(_SEQ_LEN - 1, tbuf1, osem1)


def kernel(indices, embedding_matrix):
    idx = indices.astype(jnp.int32).reshape(_NUM_WORKERS, _BBLK, _SEQ_LEN)
    idx = idx.transpose(0, 2, 1)
    out = _embedding_gather(idx, embedding_matrix)
    return out.transpose(2, 4, 0, 1, 3).reshape(_BATCH, _SEQ_LEN, _EMBED_DIM)


# speedup vs baseline: 5.7168x; 2.8775x over previous
import functools

import jax
import jax.numpy as jnp
from jax import lax
from jax.experimental import pallas as pl
from jax.experimental.pallas import tpu as pltpu
from jax.experimental.pallas import tpu_sc as plsc

_VOCAB = 100000
_EMBED_DIM = 64
_BATCH = 4096
_SEQ_LEN = 200

_NUM_WORKERS = 32
_BBLK = _BATCH // _NUM_WORKERS

_mesh = plsc.VectorSubcoreMesh(core_axis_name="c", subcore_axis_name="s")


@functools.partial(
    pl.kernel,
    mesh=_mesh,
    out_type=jax.ShapeDtypeStruct((_SEQ_LEN, 8, _NUM_WORKERS, 8, 128), jnp.float32),
    scratch_types=[
        pltpu.VMEM((_SEQ_LEN, _BBLK), jnp.int32),
        pltpu.VMEM((_BBLK, _EMBED_DIM), jnp.float32),
        pltpu.VMEM((_BBLK, _EMBED_DIM), jnp.float32),
        pltpu.VMEM((8, 8, 128), jnp.float32),
        pltpu.VMEM((8, 8, 128), jnp.float32),
        pltpu.SemaphoreType.DMA,
        pltpu.SemaphoreType.DMA,
        pltpu.SemaphoreType.DMA,
        pltpu.SemaphoreType.DMA,
    ],
    compiler_params=pltpu.CompilerParams(
        use_tc_tiling_on_sc=False, needs_layout_passes=False
    ),
)
def _embedding_gather(idx_hbm, table_hbm, out_hbm,
                      idx_v, rows0, rows1, tbuf0, tbuf1,
                      gsem0, gsem1, osem0, osem1):
    wid = lax.axis_index("s") * 2 + lax.axis_index("c")
    pltpu.sync_copy(idx_hbm.at[wid], idx_v)

    iota = jax.lax.iota(jnp.int32, 16)
    diags = [(iota + k) & 15 for k in range(16)]

    def transpose_tile(rows, tbuf):
        @plsc.parallel_loop(0, 32, 1, unroll=4)
        def stbody(st):
            dc = st >> 3
            bb = st & 7
            row = iota + bb * 16
            d0 = dc * 16
            for k in range(16):
                d = diags[k] + d0
                vals = plsc.load_gather(rows, [row, d])
                plsc.store_scatter(tbuf, [d >> 3, d & 7, row], vals)

    def gather(l, rows, sem):
        pltpu.async_copy(table_hbm.at[idx_v.at[l]], rows, sem)

    def gather_wait(l, rows, sem):
        pltpu.make_async_copy(table_hbm.at[idx_v.at[l]], rows, sem).wait()

    def store(l, tbuf, sem):
        pltpu.async_copy(tbuf, out_hbm.at[l, :, wid], sem)

    def store_wait(l, tbuf, sem):
        pltpu.make_async_copy(tbuf, out_hbm.at[l, :, wid], sem).wait()

    gather(0, rows0, gsem0)
    gather(1, rows1, gsem1)

    def body(i, carry):
        l = 2 * i
        gather_wait(l, rows0, gsem0)

        @pl.when(i > 0)
        def _():
            store_wait(l - 2, tbuf0, osem0)
        transpose_tile(rows0, tbuf0)
        store(l, tbuf0, osem0)

        @pl.when(l + 2 < _SEQ_LEN)
        def _():
            gather(l + 2, rows0, gsem0)

        gather_wait(l + 1, rows1, gsem1)

        @pl.when(i > 0)
        def _():
            store_wait(l - 1, tbuf1, osem1)
        transpose_tile(rows1, tbuf1)
        store(l + 1, tbuf1, osem1)

        @pl.when(l + 3 < _SEQ_LEN)
        def _():
            gather(l + 3, rows1, gsem1)

        return carry

    lax.fori_loop(0, _SEQ_LEN // 2, body, 0)
    store_wait(_SEQ_LEN - 2, tbuf0, osem0)
    store_wait(_SEQ_LEN - 1, tbuf1, osem1)


def kernel(indices, embedding_matrix):
    idx = indices.astype(jnp.int32).reshape(_NUM_WORKERS, _BBLK, _SEQ_LEN)
    idx = idx.transpose(0, 2, 1)
    out = _embedding_gather(idx, embedding_matrix)
    return out.transpose(2, 4, 0, 1, 3).reshape(_BATCH, _SEQ_LEN, _EMBED_DIM)


# speedup vs baseline: 5.7415x; 1.0043x over previous
import functools

import jax
import jax.numpy as jnp
from jax import lax
from jax.experimental import pallas as pl
from jax.experimental.pallas import tpu as pltpu
from jax.experimental.pallas import tpu_sc as plsc

_VOCAB = 100000
_EMBED_DIM = 64
_BATCH = 4096
_SEQ_LEN = 200

_NUM_WORKERS = 32
_BBLK = _BATCH // _NUM_WORKERS

_mesh = plsc.VectorSubcoreMesh(core_axis_name="c", subcore_axis_name="s")


@functools.partial(
    pl.kernel,
    mesh=_mesh,
    out_type=jax.ShapeDtypeStruct((_SEQ_LEN, 8, _NUM_WORKERS, 8, 128), jnp.float32),
    scratch_types=[
        pltpu.VMEM((_SEQ_LEN // 8, 8, _BBLK), jnp.int32),
        pltpu.VMEM((_BBLK, _EMBED_DIM), jnp.float32),
        pltpu.VMEM((_BBLK, _EMBED_DIM), jnp.float32),
        pltpu.VMEM((8, 8, 128), jnp.float32),
        pltpu.VMEM((8, 8, 128), jnp.float32),
        pltpu.SemaphoreType.DMA,
        pltpu.SemaphoreType.DMA,
        pltpu.SemaphoreType.DMA,
        pltpu.SemaphoreType.DMA,
    ],
    compiler_params=pltpu.CompilerParams(
        use_tc_tiling_on_sc=False, needs_layout_passes=False
    ),
)
def _embedding_gather(idx_hbm, table_hbm, out_hbm,
                      idx_v, rows0, rows1, tbuf0, tbuf1,
                      gsem0, gsem1, osem0, osem1):
    wid = lax.axis_index("s") * 2 + lax.axis_index("c")
    pltpu.sync_copy(idx_hbm.at[:, wid], idx_v)

    iota = jax.lax.iota(jnp.int32, 16)
    diags = [(iota + k) & 15 for k in range(16)]

    def transpose_tile(rows, tbuf):
        @plsc.parallel_loop(0, 32, 1, unroll=4)
        def stbody(st):
            dc = st >> 3
            bb = st & 7
            row = iota + bb * 16
            d0 = dc * 16
            for k in range(16):
                d = diags[k] + d0
                vals = plsc.load_gather(rows, [row, d])
                plsc.store_scatter(tbuf, [d >> 3, d & 7, row], vals)

    def gather(l, rows, sem):
        pltpu.async_copy(table_hbm.at[idx_v.at[l >> 3, l & 7]], rows, sem)

    def gather_wait(l, rows, sem):
        pltpu.make_async_copy(
            table_hbm.at[idx_v.at[l >> 3, l & 7]], rows, sem
        ).wait()

    def store(l, tbuf, sem):
        pltpu.async_copy(tbuf, out_hbm.at[l, :, wid], sem)

    def store_wait(l, tbuf, sem):
        pltpu.make_async_copy(tbuf, out_hbm.at[l, :, wid], sem).wait()

    gather(0, rows0, gsem0)
    gather(1, rows1, gsem1)

    def body(i, carry):
        l = 2 * i
        gather_wait(l, rows0, gsem0)

        @pl.when(i > 0)
        def _():
            store_wait(l - 2, tbuf0, osem0)
        transpose_tile(rows0, tbuf0)
        store(l, tbuf0, osem0)

        @pl.when(l + 2 < _SEQ_LEN)
        def _():
            gather(l + 2, rows0, gsem0)

        gather_wait(l + 1, rows1, gsem1)

        @pl.when(i > 0)
        def _():
            store_wait(l - 1, tbuf1, osem1)
        transpose_tile(rows1, tbuf1)
        store(l + 1, tbuf1, osem1)

        @pl.when(l + 3 < _SEQ_LEN)
        def _():
            gather(l + 3, rows1, gsem1)

        return carry

    lax.fori_loop(0, _SEQ_LEN // 2, body, 0)
    store_wait(_SEQ_LEN - 2, tbuf0, osem0)
    store_wait(_SEQ_LEN - 1, tbuf1, osem1)


def kernel(indices, embedding_matrix):
    idx = indices.astype(jnp.int32).reshape(_NUM_WORKERS, _BBLK, _SEQ_LEN // 8, 8)
    idx = idx.transpose(2, 0, 3, 1)
    out = _embedding_gather(idx, embedding_matrix)
    return out.transpose(2, 4, 0, 1, 3).reshape(_BATCH, _SEQ_LEN, _EMBED_DIM)


# speedup vs baseline: 5.9240x; 1.0318x over previous
import functools

import jax
import jax.numpy as jnp
from jax import lax
from jax.experimental import pallas as pl
from jax.experimental.pallas import tpu as pltpu
from jax.experimental.pallas import tpu_sc as plsc

_VOCAB = 100000
_EMBED_DIM = 64
_BATCH = 4096
_SEQ_LEN = 200

_NUM_WORKERS = 32
_BBLK = _BATCH // _NUM_WORKERS

_mesh = plsc.VectorSubcoreMesh(core_axis_name="c", subcore_axis_name="s")


@functools.partial(
    pl.kernel,
    mesh=_mesh,
    out_type=jax.ShapeDtypeStruct((_SEQ_LEN, 8, _NUM_WORKERS, 8, 128), jnp.float32),
    scratch_types=[
        pltpu.VMEM((_SEQ_LEN // 8, 8, _BBLK), jnp.int32),
        pltpu.VMEM((_BBLK, _EMBED_DIM), jnp.float32),
        pltpu.VMEM((_BBLK, _EMBED_DIM), jnp.float32),
        pltpu.VMEM((8, 8, 128), jnp.float32),
        pltpu.VMEM((8, 8, 128), jnp.float32),
        pltpu.SemaphoreType.DMA,
        pltpu.SemaphoreType.DMA,
        pltpu.SemaphoreType.DMA,
        pltpu.SemaphoreType.DMA,
    ],
    compiler_params=pltpu.CompilerParams(
        use_tc_tiling_on_sc=False, needs_layout_passes=False
    ),
)
def _embedding_gather(idx_hbm, table_hbm, out_hbm,
                      idx_v, rows0, rows1, tbuf0, tbuf1,
                      gsem0, gsem1, osem0, osem1):
    wid = lax.axis_index("s") * 2 + lax.axis_index("c")
    pltpu.sync_copy(idx_hbm.at[:, wid], idx_v)

    @plsc.parallel_loop(0, _SEQ_LEN, 1, unroll=4)
    def _double(st):
        lt = st >> 3
        ls = st & 7
        for c in range(8):
            v = idx_v[lt, ls, pl.ds(16 * c, 16)]
            idx_v[lt, ls, pl.ds(16 * c, 16)] = v + v

    iota = jax.lax.iota(jnp.int32, 16)
    diags = [(iota + k) & 15 for k in range(16)]

    def transpose_tile(rows, tbuf):
        @plsc.parallel_loop(0, 32, 1, unroll=4)
        def stbody(st):
            dc = st >> 3
            bb = st & 7
            row = iota + bb * 16
            d0 = dc * 16
            for k in range(16):
                d = diags[k] + d0
                vals = plsc.load_gather(rows, [row, d])
                plsc.store_scatter(tbuf, [d >> 3, d & 7, row], vals)

    def gather(l, rows, sem):
        pltpu.async_copy(table_hbm.at[idx_v.at[l >> 3, l & 7]], rows, sem)

    def gather_wait(l, rows, sem):
        pltpu.make_async_copy(
            table_hbm.at[idx_v.at[l >> 3, l & 7]], rows, sem
        ).wait()

    def store(l, tbuf, sem):
        pltpu.async_copy(tbuf, out_hbm.at[l, :, wid], sem)

    def store_wait(l, tbuf, sem):
        pltpu.make_async_copy(tbuf, out_hbm.at[l, :, wid], sem).wait()

    gather(0, rows0, gsem0)
    gather(1, rows1, gsem1)

    def body(i, carry):
        l = 2 * i
        gather_wait(l, rows0, gsem0)

        @pl.when(i > 0)
        def _():
            store_wait(l - 2, tbuf0, osem0)
        transpose_tile(rows0, tbuf0)
        store(l, tbuf0, osem0)

        @pl.when(l + 2 < _SEQ_LEN)
        def _():
            gather(l + 2, rows0, gsem0)

        gather_wait(l + 1, rows1, gsem1)

        @pl.when(i > 0)
        def _():
            store_wait(l - 1, tbuf1, osem1)
        transpose_tile(rows1, tbuf1)
        store(l + 1, tbuf1, osem1)

        @pl.when(l + 3 < _SEQ_LEN)
        def _():
            gather(l + 3, rows1, gsem1)

        return carry

    lax.fori_loop(0, _SEQ_LEN // 2, body, 0)
    store_wait(_SEQ_LEN - 2, tbuf0, osem0)
    store_wait(_SEQ_LEN - 1, tbuf1, osem1)


def kernel(indices, embedding_matrix):
    idx = indices.astype(jnp.int32).reshape(_NUM_WORKERS, _BBLK, _SEQ_LEN // 8, 8)
    idx = idx.transpose(2, 0, 3, 1)
    table = jnp.pad(embedding_matrix, ((0, 0), (0, 64)))
    table = table.reshape(_VOCAB * 2, _EMBED_DIM)
    out = _embedding_gather(idx, table)
    return out.transpose(2, 4, 0, 1, 3).reshape(_BATCH, _SEQ_LEN, _EMBED_DIM)
